# baseline (speedup 1.0000x reference)
import functools

import jax
import jax.numpy as jnp
from jax import lax
from jax.experimental import pallas as pl
from jax.experimental.pallas import tpu as pltpu

_NEG = float("-inf")
_POS = float("inf")


_RC = 8


def _accum_minmax(j, adj_ref, vvals_t, n, br, bc, smin_ref, smax_ref):
    col = lax.broadcasted_iota(jnp.int32, (1, bc), 1) + j * bc
    valid = col < n
    vmax = jnp.broadcast_to(jnp.where(valid, vvals_t, _NEG), (_RC, bc))
    vmin = jnp.broadcast_to(jnp.where(valid, vvals_t, _POS), (_RC, bc))
    for r in range(br // _RC):
        sl = slice(r * _RC, (r + 1) * _RC)
        mask = adj_ref[sl, :].astype(jnp.int32) != 0
        bmax = jnp.max(jnp.where(mask, vmax, _NEG), axis=1, keepdims=True)
        bmin = jnp.min(jnp.where(mask, vmin, _POS), axis=1, keepdims=True)
        prev_max = jnp.where(j == 0, _NEG, smax_ref[sl, :])
        prev_min = jnp.where(j == 0, _POS, smin_ref[sl, :])
        smax_ref[sl, :] = jnp.maximum(prev_max, bmax)
        smin_ref[sl, :] = jnp.minimum(prev_min, bmin)


def _substitute(smin, smax, vrow, rowvalid):
    has_nb = (smax != _NEG) & rowvalid
    mn = jnp.where(has_nb, smin, vrow)
    mx = jnp.where(has_nb, smax, vrow)
    return mn, mx


def _rowvalid(i, br, nodes_val):
    row = lax.broadcasted_iota(jnp.int32, (br, 1), 0) + i * br
    return row < nodes_val


def _layer1_body(nodes_ref, adj_ref, xt_ref, x_ref, out_ref, smin_ref,
                 smax_ref, *, n, br, bc, nj):
    i, j = pl.program_id(0), pl.program_id(1)
    _accum_minmax(j, adj_ref, xt_ref[...], n, br, bc, smin_ref, smax_ref)

    @pl.when(j == nj - 1)
    def _():
        x = x_ref[...]
        mn, mx = _substitute(smin_ref[...], smax_ref[...], x,
                             _rowvalid(i, br, nodes_ref[0, 0]))
        out_ref[...] = jnp.maximum(x - mn, mx - x)


def _layer2_body(nodes_ref, adj_ref, dt_ref, d_ref, w1_ref, b1_ref, out_ref,
                 smin_ref, smax_ref, *, n, br, bc, nj):
    i, j = pl.program_id(0), pl.program_id(1)
    w1 = w1_ref[0, 0]
    b1 = b1_ref[0, 0]
    h1_t = jnp.maximum(dt_ref[...] * w1 + b1, 0.0)
    _accum_minmax(j, adj_ref, h1_t, n, br, bc, smin_ref, smax_ref)

    @pl.when(j == nj - 1)
    def _():
        h1 = jnp.maximum(d_ref[...] * w1 + b1, 0.0)
        mn, mx = _substitute(smin_ref[...], smax_ref[...], h1,
                             _rowvalid(i, br, nodes_ref[0, 0]))
        out_ref[...] = jnp.maximum(h1 - mn, mx - h1)


def _layer3_body(nodes_ref, adj_ref, dt_ref, d_ref, w2_ref, b2_ref, w3t_ref,
                 b3_ref, out_ref, smin_ref, smax_ref, *, n, br, bc, nj):
    i, j = pl.program_id(0), pl.program_id(1)
    _accum_minmax(j, adj_ref, dt_ref[...], n, br, bc, smin_ref, smax_ref)

    @pl.when(j == nj - 1)
    def _():
        t = d_ref[...]
        tmn, tmx = _substitute(smin_ref[...], smax_ref[...], t,
                               _rowvalid(i, br, nodes_ref[0, 0]))
        w2 = w2_ref[...]
        b2 = b2_ref[...]
        pos = w2 >= 0.0
        h2 = jnp.maximum(t * w2 + b2, 0.0)
        hi_arg = jnp.where(pos, tmx, tmn)
        lo_arg = jnp.where(pos, tmn, tmx)
        h2_hi = jnp.maximum(hi_arg * w2 + b2, 0.0)
        h2_lo = jnp.maximum(lo_arg * w2 + b2, 0.0)
        dev3 = jnp.maximum(h2 - h2_lo, h2_hi - h2)
        z = jnp.sum(dev3 * w3t_ref[...], axis=1, keepdims=True) + b3_ref[0, 0]
        out_ref[...] = 1.0 / (1.0 + jnp.exp(-z))


def _run_layer(body, adj, vt, vrow, params, nodes_arr, br, bc):
    n = adj.shape[0]
    ni, nj = pl.cdiv(n, br), pl.cdiv(n, bc)
    scalar_specs = [pl.BlockSpec(p.shape, lambda i, j: (0, 0))
                    for p in params]
    return pl.pallas_call(
        functools.partial(body, n=n, br=br, bc=bc, nj=nj),
        grid=(ni, nj),
        in_specs=[
            pl.BlockSpec((1, 1), lambda i, j: (0, 0)),
            pl.BlockSpec((br, bc), lambda i, j: (i, j)),
            pl.BlockSpec((1, bc), lambda i, j: (0, j)),
            pl.BlockSpec((br, 1), lambda i, j: (i, 0)),
        ] + scalar_specs,
        out_specs=pl.BlockSpec((br, 1), lambda i, j: (i, 0)),
        out_shape=jax.ShapeDtypeStruct((n, 1), jnp.float32),
        scratch_shapes=[pltpu.VMEM((br, 1), jnp.float32),
                        pltpu.VMEM((br, 1), jnp.float32)],
    )(nodes_arr, adj, vt, vrow, *params)


def _gnn(x, nodes, adjacency_matrix, W1, b1, W2, b2, W3, b3, br, bc):
    n = x.shape[0]
    nodes_arr = jnp.asarray(nodes, jnp.int32).reshape(1, 1)
    w1 = W1.reshape(1, 1)
    b1v = b1.reshape(1, 1)
    w2 = W2.reshape(1, 64)
    b2v = b2.reshape(1, 64)
    w3t = W3.reshape(1, 64)
    b3v = b3.reshape(1, 1)

    adj8 = adjacency_matrix.view(jnp.int8)
    dev1 = _run_layer(_layer1_body, adj8, x.reshape(1, n), x,
                      [], nodes_arr, br, bc)
    dev2 = _run_layer(_layer2_body, adj8, dev1.reshape(1, n),
                      dev1, [w1, b1v], nodes_arr, br, bc)
    y = _run_layer(_layer3_body, adj8, dev2.reshape(1, n),
                   dev2, [w2, b2v, w3t, b3v], nodes_arr, br, bc)
    return y


def kernel(x, nodes, adjacency_matrix, W1, b1, W2, b2, W3, b3):
    return _gnn(x, nodes, adjacency_matrix, W1, b1, W2, b2, W3, b3,
                br=2000, bc=2560)


# speedup vs baseline: 1.1437x; 1.1437x over previous
import functools

import jax
import jax.numpy as jnp
from jax import lax
from jax.experimental import pallas as pl
from jax.experimental.pallas import tpu as pltpu

_NEG = float("-inf")
_POS = float("inf")

_RC = 8


def _accum_minmax(j, adj_ref, vvals_t, n, br, bc, smin_ref, smax_ref):
    col = lax.broadcasted_iota(jnp.int32, (1, bc), 1) + j * bc
    valid = col < n
    vmax = jnp.broadcast_to(jnp.where(valid, vvals_t, _NEG), (_RC, bc))
    vmin = jnp.broadcast_to(jnp.where(valid, vvals_t, _POS), (_RC, bc))
    for r in range(br // _RC):
        sl = slice(r * _RC, (r + 1) * _RC)
        mask = adj_ref[sl, :].astype(jnp.int32) != 0
        bmax = jnp.max(jnp.where(mask, vmax, _NEG), axis=1, keepdims=True)
        bmin = jnp.min(jnp.where(mask, vmin, _POS), axis=1, keepdims=True)
        prev_max = jnp.where(j == 0, _NEG, smax_ref[sl, :])
        prev_min = jnp.where(j == 0, _POS, smin_ref[sl, :])
        smax_ref[sl, :] = jnp.maximum(prev_max, bmax)
        smin_ref[sl, :] = jnp.minimum(prev_min, bmin)


def _substitute(smin, smax, vrow, rowvalid):
    has_nb = (smax != _NEG) & rowvalid
    mn = jnp.where(has_nb, smin, vrow)
    mx = jnp.where(has_nb, smax, vrow)
    return mn, mx


def _fused_body(nodes_ref, adj_ref, xt_ref, x_ref, w1_ref, b1_ref, w2_ref,
                b2_ref, w3t_ref, b3_ref, out_ref, smin_ref, smax_ref,
                dta_ref, dtb_ref, ds_ref, *, n, br, bc, nj):
    l, i, j = pl.program_id(0), pl.program_id(1), pl.program_id(2)

    va = dta_ref[0:1, pl.ds(j * bc, bc)]
    vb = dtb_ref[0:1, pl.ds(j * bc, bc)]
    vraw = jnp.where(l == 0, xt_ref[...], jnp.where(l == 1, va, vb))
    _accum_minmax(j, adj_ref, vraw, n, br, bc, smin_ref, smax_ref)

    row = lax.broadcasted_iota(jnp.int32, (br, 1), 0) + i * br
    rowvalid = row < nodes_ref[0, 0]

    @pl.when((j == nj - 1) & (l == 0))
    def _():
        x = x_ref[...]
        mn, mx = _substitute(smin_ref[...], smax_ref[...], x, rowvalid)
        dev1 = jnp.maximum(x - mn, mx - x)
        h1 = jnp.maximum(dev1 * w1_ref[0, 0] + b1_ref[0, 0], 0.0)
        ds_ref[pl.ds(i * br, br), :] = h1
        dta_ref[0:1, pl.ds(i * br, br)] = jnp.reshape(h1, (1, br))

    @pl.when((j == nj - 1) & (l == 1))
    def _():
        h1 = ds_ref[pl.ds(i * br, br), :]
        mn, mx = _substitute(smin_ref[...], smax_ref[...], h1, rowvalid)
        dev2 = jnp.maximum(h1 - mn, mx - h1)
        ds_ref[pl.ds(i * br, br), :] = dev2
        dtb_ref[0:1, pl.ds(i * br, br)] = jnp.reshape(dev2, (1, br))

    @pl.when((j == nj - 1) & (l == 2))
    def _():
        t = ds_ref[pl.ds(i * br, br), :]
        tmn, tmx = _substitute(smin_ref[...], smax_ref[...], t, rowvalid)
        w2 = w2_ref[...]
        b2 = b2_ref[...]
        pos = w2 >= 0.0
        h2 = jnp.maximum(t * w2 + b2, 0.0)
        hi_arg = jnp.where(pos, tmx, tmn)
        lo_arg = jnp.where(pos, tmn, tmx)
        h2_hi = jnp.maximum(hi_arg * w2 + b2, 0.0)
        h2_lo = jnp.maximum(lo_arg * w2 + b2, 0.0)
        dev3 = jnp.maximum(h2 - h2_lo, h2_hi - h2)
        z = jnp.sum(dev3 * w3t_ref[...], axis=1, keepdims=True) + b3_ref[0, 0]
        out_ref[...] = 1.0 / (1.0 + jnp.exp(-z))


def _gnn(x, nodes, adjacency_matrix, W1, b1, W2, b2, W3, b3, br, bc):
    n = x.shape[0]
    ni, nj = pl.cdiv(n, br), pl.cdiv(n, bc)
    npad = max(ni * br, nj * bc)
    nodes_arr = jnp.asarray(nodes, jnp.int32).reshape(1, 1)
    w1 = W1.reshape(1, 1)
    b1v = b1.reshape(1, 1)
    w2 = W2.reshape(1, 64)
    b2v = b2.reshape(1, 64)
    w3t = W3.reshape(1, 64)
    b3v = b3.reshape(1, 1)
    adj8 = adjacency_matrix.view(jnp.int8)

    return pl.pallas_call(
        functools.partial(_fused_body, n=n, br=br, bc=bc, nj=nj),
        grid=(3, ni, nj),
        in_specs=[
            pl.BlockSpec((1, 1), lambda l, i, j: (0, 0)),
            pl.BlockSpec((br, bc), lambda l, i, j: (i, j)),
            pl.BlockSpec((1, bc), lambda l, i, j: (0, j)),
            pl.BlockSpec((br, 1), lambda l, i, j: (i, 0)),
            pl.BlockSpec((1, 1), lambda l, i, j: (0, 0)),
            pl.BlockSpec((1, 1), lambda l, i, j: (0, 0)),
            pl.BlockSpec((1, 64), lambda l, i, j: (0, 0)),
            pl.BlockSpec((1, 64), lambda l, i, j: (0, 0)),
            pl.BlockSpec((1, 64), lambda l, i, j: (0, 0)),
            pl.BlockSpec((1, 1), lambda l, i, j: (0, 0)),
        ],
        out_specs=pl.BlockSpec((br, 1), lambda l, i, j: (i, 0)),
        out_shape=jax.ShapeDtypeStruct((n, 1), jnp.float32),
        scratch_shapes=[pltpu.VMEM((br, 1), jnp.float32),
                        pltpu.VMEM((br, 1), jnp.float32),
                        pltpu.VMEM((1, npad), jnp.float32),
                        pltpu.VMEM((1, npad), jnp.float32),
                        pltpu.VMEM((npad, 1), jnp.float32)],
    )(nodes_arr, adj8, x.reshape(1, n), x, w1, b1v, w2, b2v, w3t, b3v)


def kernel(x, nodes, adjacency_matrix, W1, b1, W2, b2, W3, b3):
    return _gnn(x, nodes, adjacency_matrix, W1, b1, W2, b2, W3, b3,
                br=2048, bc=2048)
